# baseline (speedup 1.0000x reference)
import jax
import jax.numpy as jnp
from jax.experimental import pallas as pl
from jax.experimental.pallas import tpu as pltpu

_BM = 400
_NBUF = 3


def _rgcn_body(adj_ref, x_ref, w_ref, b_ref, o_ref, buf, tbuf, acc,
               sems, tsem):
    n = x_ref.shape[0]
    nrel = adj_ref.shape[0]
    nfull = n // _BM
    ntail = n - nfull * _BM
    nslab = nrel * nfull

    def _copy(s, slot):
        r = jax.lax.rem(s, nrel)
        m = jax.lax.div(s, nrel)
        return pltpu.make_async_copy(
            adj_ref.at[r, pl.ds(pl.multiple_of(m * _BM, 8), _BM), :],
            buf.at[slot],
            sems.at[slot],
        )

    def _tail_copy(r):
        return pltpu.make_async_copy(
            adj_ref.at[r, pl.ds(nfull * _BM, ntail), :],
            tbuf.at[r, pl.ds(0, ntail), :],
            tsem,
        )

    for s0 in range(min(_NBUF, nslab)):
        _copy(jnp.int32(s0), jnp.int32(s0)).start()
    if ntail:
        for r0 in range(nrel):
            _tail_copy(r0).start()

    def _step(s, carry):
        slot = jax.lax.rem(s, _NBUF)
        r = jax.lax.rem(s, nrel)
        m = jax.lax.div(s, nrel)
        _copy(s, slot).wait()
        msg = jax.lax.dot(buf[slot], x_ref[...],
                          preferred_element_type=jnp.float32)
        part = jax.lax.dot(msg, w_ref[r], preferred_element_type=jnp.float32)

        @pl.when(r == 0)
        def _first():
            acc[...] = part

        @pl.when(r == nrel - 1)
        def _last():
            row = pl.multiple_of(m * _BM, 8)
            o_ref[pl.ds(row, _BM), :] = jnp.maximum(
                acc[...] + part + b_ref[...], 0.0)

        @pl.when(s + _NBUF < nslab)
        def _refill():
            _copy(s + _NBUF, slot).start()

        return carry

    jax.lax.fori_loop(0, nslab, _step, 0)

    if ntail:
        for r0 in range(nrel):
            _tail_copy(r0).wait()
        tout = b_ref[...]
        for r0 in range(nrel):
            tmsg = jax.lax.dot(tbuf[r0], x_ref[...],
                               preferred_element_type=jnp.float32)
            tout = tout + jax.lax.dot(tmsg, w_ref[r0],
                                      preferred_element_type=jnp.float32)
        o_ref[pl.ds(nfull * _BM, ntail), :] = jnp.maximum(tout[:ntail], 0.0)


def kernel(node_features, adj_list, weight, bias):
    n, in_dim = node_features.shape
    r = adj_list.shape[0]
    out_dim = weight.shape[-1]
    ntail = n % _BM

    b2 = bias.reshape(1, out_dim)

    return pl.pallas_call(
        _rgcn_body,
        in_specs=[
            pl.BlockSpec(memory_space=pl.ANY),
            pl.BlockSpec(memory_space=pltpu.VMEM),
            pl.BlockSpec(memory_space=pltpu.VMEM),
            pl.BlockSpec(memory_space=pltpu.VMEM),
        ],
        out_specs=pl.BlockSpec(memory_space=pltpu.VMEM),
        out_shape=jax.ShapeDtypeStruct((n, out_dim), jnp.float32),
        scratch_shapes=[
            pltpu.VMEM((_NBUF, _BM, n), jnp.float32),
            pltpu.VMEM((r, max(ntail, 8), n), jnp.float32),
            pltpu.VMEM((_BM, out_dim), jnp.float32),
            pltpu.SemaphoreType.DMA((_NBUF,)),
            pltpu.SemaphoreType.DMA,
        ],
    )(adj_list, node_features, weight, b2)


# speedup vs baseline: 1.0158x; 1.0158x over previous
import jax
import jax.numpy as jnp
from jax.experimental import pallas as pl
from jax.experimental.pallas import tpu as pltpu

_BM = 200
_NBUF = 5


def _rgcn_body(adj_ref, x_ref, w_ref, b_ref, o_ref, buf, acc, sems):
    n = x_ref.shape[0]
    nrel = adj_ref.shape[0]
    nslab = nrel * (n // _BM)

    def _copy(s, slot):
        r = jax.lax.rem(s, nrel)
        m = jax.lax.div(s, nrel)
        return pltpu.make_async_copy(
            adj_ref.at[r, pl.ds(pl.multiple_of(m * _BM, 8), _BM), :],
            buf.at[slot],
            sems.at[slot],
        )

    for s0 in range(_NBUF - 1):
        _copy(jnp.int32(s0), jnp.int32(s0)).start()

    def _step(s, carry):
        slot = jax.lax.rem(s, _NBUF)

        @pl.when(s + _NBUF - 1 < nslab)
        def _refill():
            _copy(s + _NBUF - 1, jax.lax.rem(s + _NBUF - 1, _NBUF)).start()

        r = jax.lax.rem(s, nrel)
        m = jax.lax.div(s, nrel)
        _copy(s, slot).wait()
        msg = jax.lax.dot(buf[slot], x_ref[...],
                          preferred_element_type=jnp.float32)
        part = jax.lax.dot(msg, w_ref[r], preferred_element_type=jnp.float32)

        @pl.when(r == 0)
        def _first():
            acc[...] = part

        @pl.when(r == nrel - 1)
        def _last():
            row = pl.multiple_of(m * _BM, 8)
            o_ref[pl.ds(row, _BM), :] = jnp.maximum(
                acc[...] + part + b_ref[...], 0.0)

        return carry

    jax.lax.fori_loop(0, nslab, _step, 0)


def kernel(node_features, adj_list, weight, bias):
    n, in_dim = node_features.shape
    r = adj_list.shape[0]
    out_dim = weight.shape[-1]

    b2 = bias.reshape(1, out_dim)

    return pl.pallas_call(
        _rgcn_body,
        in_specs=[
            pl.BlockSpec(memory_space=pl.ANY),
            pl.BlockSpec(memory_space=pltpu.VMEM),
            pl.BlockSpec(memory_space=pltpu.VMEM),
            pl.BlockSpec(memory_space=pltpu.VMEM),
        ],
        out_specs=pl.BlockSpec(memory_space=pltpu.VMEM),
        out_shape=jax.ShapeDtypeStruct((n, out_dim), jnp.float32),
        scratch_shapes=[
            pltpu.VMEM((_NBUF, _BM, n), jnp.float32),
            pltpu.VMEM((_BM, out_dim), jnp.float32),
            pltpu.SemaphoreType.DMA((_NBUF,)),
        ],
    )(adj_list, node_features, weight, b2)


# speedup vs baseline: 1.0213x; 1.0053x over previous
import jax
import jax.numpy as jnp
from jax.experimental import pallas as pl
from jax.experimental.pallas import tpu as pltpu

_BM = 200
_NBUF = 4


def _rgcn_body(adj_ref, x_ref, w_ref, b_ref, o_ref, buf, acc, sems):
    n = x_ref.shape[0]
    nrel = adj_ref.shape[0]
    nslab = nrel * (n // _BM)

    def _copy(s, slot):
        r = jax.lax.rem(s, nrel)
        m = jax.lax.div(s, nrel)
        return pltpu.make_async_copy(
            adj_ref.at[r, pl.ds(pl.multiple_of(m * _BM, 8), _BM), :],
            buf.at[slot],
            sems.at[slot],
        )

    for s0 in range(_NBUF):
        _copy(jnp.int32(s0), jnp.int32(s0)).start()

    def _step(s, carry):
        slot = jax.lax.rem(s, _NBUF)
        r = jax.lax.rem(s, nrel)
        m = jax.lax.div(s, nrel)
        _copy(s, slot).wait()
        msg = jax.lax.dot(buf[slot], x_ref[...],
                          preferred_element_type=jnp.float32)
        part = jax.lax.dot(msg, w_ref[r], preferred_element_type=jnp.float32)

        @pl.when(r == 0)
        def _first():
            acc[...] = part

        @pl.when(r == nrel - 1)
        def _last():
            row = pl.multiple_of(m * _BM, 8)
            o_ref[pl.ds(row, _BM), :] = jnp.maximum(
                acc[...] + part + b_ref[...], 0.0)

        @pl.when(s + _NBUF < nslab)
        def _refill():
            _copy(s + _NBUF, slot).start()

        return carry

    jax.lax.fori_loop(0, nslab, _step, 0)


def kernel(node_features, adj_list, weight, bias):
    n, in_dim = node_features.shape
    r = adj_list.shape[0]
    out_dim = weight.shape[-1]

    b2 = bias.reshape(1, out_dim)

    return pl.pallas_call(
        _rgcn_body,
        in_specs=[
            pl.BlockSpec(memory_space=pl.ANY),
            pl.BlockSpec(memory_space=pltpu.VMEM),
            pl.BlockSpec(memory_space=pltpu.VMEM),
            pl.BlockSpec(memory_space=pltpu.VMEM),
        ],
        out_specs=pl.BlockSpec(memory_space=pltpu.VMEM),
        out_shape=jax.ShapeDtypeStruct((n, out_dim), jnp.float32),
        scratch_shapes=[
            pltpu.VMEM((_NBUF, _BM, n), jnp.float32),
            pltpu.VMEM((_BM, out_dim), jnp.float32),
            pltpu.SemaphoreType.DMA((_NBUF,)),
        ],
    )(adj_list, node_features, weight, b2)
